# baseline (speedup 1.0000x reference)
import dataclasses
import functools

import jax
import jax.numpy as jnp
from jax import lax
from jax.experimental import pallas as pl
from jax.experimental.pallas import tpu as pltpu
from jax.experimental.pallas import tpu_sc as plsc

_N_USERS = 20000
_N_ITEMS = 29998
_EMB = 64
_N = (_N_USERS + 1) + (_N_ITEMS + 1)
_E = 800000
_REG_WEIGHT = 0.001

_HALF = 25000
_HALF_PAD = 25600
_PADN = 2 * _HALF_PAD
_C = 320
_CHUNKS = _E // _C
_NS = 16
_ROWS_PER_TILE = _HALF_PAD // _NS


def _propagate_layer(x_pad, src_pad, dst, vals):
    mesh = plsc.VectorSubcoreMesh(core_axis_name="c", subcore_axis_name="s")
    cp = pltpu.CompilerParams()
    if "needs_layout_passes" in pltpu.CompilerParams.__dataclass_fields__:
        cp = dataclasses.replace(cp, needs_layout_passes=False)
    if "use_tc_tiling_on_sc" in pltpu.CompilerParams.__dataclass_fields__:
        cp = dataclasses.replace(cp, use_tc_tiling_on_sc=False)

    @functools.partial(
        pl.kernel,
        compiler_params=cp,
        out_type=jax.ShapeDtypeStruct((_PADN, _EMB), jnp.float32),
        mesh=mesh,
        scratch_types=[
            pltpu.VMEM_SHARED((_HALF_PAD, _EMB), jnp.float32),
            pltpu.VMEM((_C,), jnp.int32),
            pltpu.VMEM((_C,), jnp.int32),
            pltpu.VMEM((_C,), jnp.float32),
            pltpu.VMEM((_C,), jnp.int32),
            pltpu.VMEM((_C, _EMB), jnp.float32),
            pltpu.SemaphoreType.DMA,
        ],
    )
    def layer(x_hbm, src_hbm, dst_hbm, vals_hbm, y_hbm,
              acc, src_v, dst_v, vals_v, idx_v, rows_v, sem):
        c = lax.axis_index("c")
        s = lax.axis_index("s")
        zeros16 = jnp.zeros((16,), jnp.float32)

        @pl.loop(0, _C)
        def _(r):
            for q in range(_EMB // 16):
                rows_v[r, pl.ds(16 * q, 16)] = zeros16

        tile_base = s * _ROWS_PER_TILE

        @pl.loop(0, _ROWS_PER_TILE // _C)
        def _(t):
            pltpu.sync_copy(rows_v, acc.at[pl.ds(tile_base + t * _C, _C)])

        plsc.subcore_barrier()

        half_base = c * _HALF
        lane = lax.iota(jnp.int32, 16)

        @pl.loop(0, (_CHUNKS + _NS - 1) // _NS)
        def _(k):
            chunk = k * _NS + s

            @pl.when(chunk < _CHUNKS)
            def _():
                base = chunk * _C
                pltpu.sync_copy(src_hbm.at[pl.ds(base, _C)], src_v)
                pltpu.sync_copy(dst_hbm.at[pl.ds(base, _C)], dst_v)
                pltpu.sync_copy(vals_hbm.at[pl.ds(base, _C)], vals_v)
                pltpu.async_copy(x_hbm.at[src_v], rows_v, sem).wait()

                @plsc.parallel_loop(0, _C, step=16)
                def _(i):
                    d = dst_v[pl.ds(i, 16)]
                    dl = d - half_base
                    inr = (dl >= 0) & (dl < _HALF)
                    dump = _HALF + (i & 0x1F0) + lane
                    idx_v[pl.ds(i, 16)] = jnp.where(inr, dl, dump)
                    v16 = vals_v[pl.ds(i, 16)]
                    for j in range(16):
                        vb = jnp.broadcast_to(v16[j], (16,))
                        for q in range(_EMB // 16):
                            sl = pl.ds(16 * q, 16)
                            rows_v[i + j, sl] = rows_v[i + j, sl] * vb

                pltpu.sync_copy(rows_v, acc.at[idx_v], add=True)

        plsc.subcore_barrier()
        pltpu.sync_copy(
            acc.at[pl.ds(tile_base, _ROWS_PER_TILE)],
            y_hbm.at[pl.ds(c * _HALF_PAD + tile_base, _ROWS_PER_TILE)])

    return layer(x_pad, src_pad, dst, vals)


def _to_pad(ids):
    return ids + jnp.where(ids >= _HALF, _HALF_PAD - _HALF, 0).astype(ids.dtype)


def kernel(user_emb, item_emb, edge_vals, trans_mat, edge_src, edge_dst,
           batch_data):
    x0 = jnp.concatenate([user_emb, item_emb], axis=0)
    pad = jnp.zeros((_HALF_PAD - _HALF, _EMB), jnp.float32)
    x0p = jnp.concatenate([x0[:_HALF], pad, x0[_HALF:], pad], axis=0)

    src_pad = _to_pad(edge_src.astype(jnp.int32))
    dst = edge_dst.astype(jnp.int32)
    vals = edge_vals.astype(jnp.float32)

    x1p = _propagate_layer(x0p, src_pad, dst, vals)
    x2p = _propagate_layer(x1p, src_pad, dst, vals)

    users = batch_data[:, 0, 0].astype(jnp.int32)
    items = batch_data[:, 0, 1:].astype(jnp.int32)
    unodes = users
    inodes = _N_USERS + 1 + items
    nodes = jnp.concatenate([unodes[:, None], inodes], axis=1)
    pnodes = _to_pad(nodes)

    x0r = x0p[pnodes]
    x1r = x1p[pnodes]
    x2r = x2p[pnodes]
    tmr = trans_mat[nodes]

    gcn = (x0r + x1r + x2r) / 3.0
    post = tmr * gcn
    norm = jnp.linalg.norm(post, axis=-1, keepdims=True)
    cur = post / jnp.maximum(norm, 1e-12) + x0r

    uf = cur[:, 0, :]
    scores = jnp.sum(uf[:, None, :] * cur[:, 1:, :], axis=2)
    mask = users != 0
    bpr = -jnp.log(jax.nn.sigmoid(scores[:, 0] - scores[:, 1]) + 1e-10)
    bpr_loss = jnp.where(mask, bpr, 0.0).sum() / jnp.maximum(mask.sum(), 1)

    emb_loss = (jnp.linalg.norm(user_emb) + jnp.linalg.norm(item_emb)) \
        / item_emb.shape[0]
    return bpr_loss + _REG_WEIGHT * emb_loss


# speedup vs baseline: 1.5931x; 1.5931x over previous
import dataclasses
import functools

import jax
import jax.numpy as jnp
from jax import lax
from jax.experimental import pallas as pl
from jax.experimental.pallas import tpu as pltpu
from jax.experimental.pallas import tpu_sc as plsc

_N_USERS = 20000
_N_ITEMS = 29998
_EMB = 64
_N = (_N_USERS + 1) + (_N_ITEMS + 1)
_E = 800000
_REG_WEIGHT = 0.001

_HALF = 25000
_HALF_PAD = 25600
_PADN = 2 * _HALF_PAD
_C = 160
_SBE = 1600
_NCH = _SBE // _C
_SBS = _E // _SBE
_NS = 16
_ROWS_PER_TILE = _HALF_PAD // _NS


def _propagate_layer(x_pad, src_pad, dst, vals):
    mesh = plsc.VectorSubcoreMesh(core_axis_name="c", subcore_axis_name="s")
    cp = pltpu.CompilerParams()
    if "needs_layout_passes" in pltpu.CompilerParams.__dataclass_fields__:
        cp = dataclasses.replace(cp, needs_layout_passes=False)
    if "use_tc_tiling_on_sc" in pltpu.CompilerParams.__dataclass_fields__:
        cp = dataclasses.replace(cp, use_tc_tiling_on_sc=False)

    @functools.partial(
        pl.kernel,
        compiler_params=cp,
        out_type=jax.ShapeDtypeStruct((_PADN, _EMB), jnp.float32),
        mesh=mesh,
        scratch_types=[
            pltpu.VMEM_SHARED((_HALF_PAD, _EMB), jnp.float32),
            pltpu.VMEM((_SBE,), jnp.int32),
            pltpu.VMEM((_SBE,), jnp.int32),
            pltpu.VMEM((_SBE,), jnp.float32),
            pltpu.VMEM((_C,), jnp.int32),
            pltpu.VMEM((_C, _EMB), jnp.float32),
            pltpu.VMEM((_C, _EMB), jnp.float32),
            pltpu.SemaphoreType.DMA,
            pltpu.SemaphoreType.DMA,
        ],
    )
    def layer(x_hbm, src_hbm, dst_hbm, vals_hbm, y_hbm,
              acc, src_v, dst_v, vals_v, idx_v, rows0, rows1, sem0, sem1):
        c = lax.axis_index("c")
        s = lax.axis_index("s")
        zeros16 = jnp.zeros((16,), jnp.float32)

        @pl.loop(0, _C)
        def _(r):
            for q in range(_EMB // 16):
                rows0[r, pl.ds(16 * q, 16)] = zeros16

        tile_base = s * _ROWS_PER_TILE

        @pl.loop(0, _ROWS_PER_TILE // _C)
        def _(t):
            pltpu.sync_copy(rows0, acc.at[pl.ds(tile_base + t * _C, _C)])

        plsc.subcore_barrier()

        half_base = c * _HALF
        lane = lax.iota(jnp.int32, 16)

        def compute_and_scatter(t, rows_b):
            coff = t * _C

            @plsc.parallel_loop(0, _C, step=16)
            def _(i):
                d = dst_v[pl.ds(coff + i, 16)]
                dl = d - half_base
                inr = (dl >= 0) & (dl < _HALF)
                dump = _HALF + (i & 0x1F0) + lane
                idx_v[pl.ds(i, 16)] = jnp.where(inr, dl, dump)
                v16 = vals_v[pl.ds(coff + i, 16)]
                for j in range(16):
                    vb = jnp.broadcast_to(v16[j], (16,))
                    for q in range(_EMB // 16):
                        sl = pl.ds(16 * q, 16)
                        rows_b[i + j, sl] = rows_b[i + j, sl] * vb

            pltpu.sync_copy(rows_b, acc.at[idx_v], add=True)

        def gather(t, rows_b, sem_b):
            return pltpu.async_copy(
                x_hbm.at[src_v.at[pl.ds(t * _C, _C)]], rows_b, sem_b)

        def gather_wait(t, rows_b, sem_b):
            pltpu.make_async_copy(
                x_hbm.at[src_v.at[pl.ds(t * _C, _C)]], rows_b, sem_b).wait()

        @pl.loop(0, (_SBS + _NS - 1) // _NS)
        def _(k):
            sb = k * _NS + s

            @pl.when(sb < _SBS)
            def _():
                sbase = sb * _SBE
                pltpu.sync_copy(src_hbm.at[pl.ds(sbase, _SBE)], src_v)
                pltpu.sync_copy(dst_hbm.at[pl.ds(sbase, _SBE)], dst_v)
                pltpu.sync_copy(vals_hbm.at[pl.ds(sbase, _SBE)], vals_v)
                gather(0, rows0, sem0)

                @pl.loop(0, _NCH // 2)
                def _(g):
                    t0 = g * 2
                    gather(t0 + 1, rows1, sem1)
                    gather_wait(t0, rows0, sem0)
                    compute_and_scatter(t0, rows0)

                    @pl.when(g < _NCH // 2 - 1)
                    def _():
                        gather(t0 + 2, rows0, sem0)

                    gather_wait(t0 + 1, rows1, sem1)
                    compute_and_scatter(t0 + 1, rows1)

        plsc.subcore_barrier()
        pltpu.sync_copy(
            acc.at[pl.ds(tile_base, _ROWS_PER_TILE)],
            y_hbm.at[pl.ds(c * _HALF_PAD + tile_base, _ROWS_PER_TILE)])

    return layer(x_pad, src_pad, dst, vals)


def _to_pad(ids):
    return ids + jnp.where(ids >= _HALF, _HALF_PAD - _HALF, 0).astype(ids.dtype)


def kernel(user_emb, item_emb, edge_vals, trans_mat, edge_src, edge_dst,
           batch_data):
    x0 = jnp.concatenate([user_emb, item_emb], axis=0)
    pad = jnp.zeros((_HALF_PAD - _HALF, _EMB), jnp.float32)
    x0p = jnp.concatenate([x0[:_HALF], pad, x0[_HALF:], pad], axis=0)

    src_pad = _to_pad(edge_src.astype(jnp.int32))
    dst = edge_dst.astype(jnp.int32)
    vals = edge_vals.astype(jnp.float32)

    x1p = _propagate_layer(x0p, src_pad, dst, vals)
    x2p = _propagate_layer(x1p, src_pad, dst, vals)

    users = batch_data[:, 0, 0].astype(jnp.int32)
    items = batch_data[:, 0, 1:].astype(jnp.int32)
    unodes = users
    inodes = _N_USERS + 1 + items
    nodes = jnp.concatenate([unodes[:, None], inodes], axis=1)
    pnodes = _to_pad(nodes)

    x0r = x0p[pnodes]
    x1r = x1p[pnodes]
    x2r = x2p[pnodes]
    tmr = trans_mat[nodes]

    gcn = (x0r + x1r + x2r) / 3.0
    post = tmr * gcn
    norm = jnp.linalg.norm(post, axis=-1, keepdims=True)
    cur = post / jnp.maximum(norm, 1e-12) + x0r

    uf = cur[:, 0, :]
    scores = jnp.sum(uf[:, None, :] * cur[:, 1:, :], axis=2)
    mask = users != 0
    bpr = -jnp.log(jax.nn.sigmoid(scores[:, 0] - scores[:, 1]) + 1e-10)
    bpr_loss = jnp.where(mask, bpr, 0.0).sum() / jnp.maximum(mask.sum(), 1)

    emb_loss = (jnp.linalg.norm(user_emb) + jnp.linalg.norm(item_emb)) \
        / item_emb.shape[0]
    return bpr_loss + _REG_WEIGHT * emb_loss
